# baseline (speedup 1.0000x reference)
import functools

import jax
import jax.numpy as jnp
from jax import lax
from jax.experimental import pallas as pl
from jax.experimental.pallas import tpu as pltpu
from jax.experimental.pallas import tpu_sc as plsc

EMB = 64
FIELDS = 26
BATCH = 16384
NBLK = BATCH // 128
NC = 2
NS = 16
NW = NC * NS
CPW = NBLK // NW
UPW = CPW * FIELDS
NROWS = 1000000
TCOL = 512


def _tc_in_body(x_ref, o_ref):
    xt = x_ref[...].T
    o_ref[...] = jnp.concatenate(
        [xt, jnp.zeros((TCOL, 128 - EMB), jnp.float32)], axis=1)


@jax.jit
def _tc_in(wt):
    grid = (NROWS + TCOL - 1) // TCOL
    return pl.pallas_call(
        _tc_in_body,
        grid=(grid,),
        in_specs=[pl.BlockSpec((EMB, TCOL), lambda i: (0, i))],
        out_specs=pl.BlockSpec((TCOL, 128), lambda i: (i, 0)),
        out_shape=jax.ShapeDtypeStruct((NROWS, 128), jnp.float32),
    )(wt)


def _tc_out_body(x_ref, o_ref):
    o_ref[...] = lax.slice(x_ref[...].T, (0, 0), (EMB, 128))


@jax.jit
def _tc_out(stage):
    return pl.pallas_call(
        _tc_out_body,
        grid=(FIELDS, NBLK),
        in_specs=[pl.BlockSpec((128, 128), lambda f, c: (f * NBLK + c, 0))],
        out_specs=pl.BlockSpec((EMB, 128), lambda f, c: (f, c)),
        out_shape=jax.ShapeDtypeStruct((FIELDS * EMB, BATCH), jnp.float32),
    )(stage)


@jax.jit
def _sc_gather(w128, idx_t):
    mesh = plsc.VectorSubcoreMesh(core_axis_name="c", subcore_axis_name="s")

    @functools.partial(
        pl.kernel,
        out_type=jax.ShapeDtypeStruct((FIELDS * BATCH, 128), jnp.float32),
        mesh=mesh,
        scratch_types=(
            pltpu.VMEM((UPW, 128), jnp.int32),
            pltpu.VMEM((128,), jnp.int32),
            pltpu.VMEM((128,), jnp.int32),
            pltpu.VMEM((128, 128), jnp.float32),
            pltpu.VMEM((128, 128), jnp.float32),
            pltpu.SemaphoreType.DMA,
            pltpu.SemaphoreType.DMA,
            pltpu.SemaphoreType.DMA,
            pltpu.SemaphoreType.DMA,
            pltpu.SemaphoreType.DMA,
        ),
        compiler_params=pltpu.CompilerParams(
            use_tc_tiling_on_sc=True, needs_layout_passes=False),
    )
    def k(w_hbm, idx_hbm, out_hbm, idxall, ilist0, ilist1,
          gbuf0, gbuf1, i_sem, g_s0, g_s1, o_s0, o_s1):
        ilist = (ilist0, ilist1)
        gbuf = (gbuf0, gbuf1)
        g_s = (g_s0, g_s1)
        o_s = (o_s0, o_s1)
        wid = lax.axis_index("s") * NC + lax.axis_index("c")

        for ci in range(CPW):
            col = (wid + NW * ci) * 128
            pltpu.async_copy(
                idx_hbm.at[pl.ds(0, FIELDS), pl.ds(col, 128)],
                idxall.at[pl.ds(ci * FIELDS, FIELDS)], i_sem)
        for ci in range(CPW):
            col = (wid + NW * ci) * 128
            pltpu.make_async_copy(
                idx_hbm.at[pl.ds(0, FIELDS), pl.ds(col, 128)],
                idxall.at[pl.ds(ci * FIELDS, FIELDS)], i_sem).wait()

        def unit_fc(t):
            t = jnp.asarray(t, jnp.int32)
            ci = ((t >= FIELDS).astype(jnp.int32)
                  + (t >= 2 * FIELDS).astype(jnp.int32)
                  + (t >= 3 * FIELDS).astype(jnp.int32))
            f = t - FIELDS * ci
            return ci, f

        def out_slice(t):
            ci, f = unit_fc(t)
            col = (wid + NW * ci) * 128
            return out_hbm.at[pl.ds(f * BATCH + col, 128), pl.ds(0, 128)]

        def stage(t, b):
            for g in range(8):
                iv = idxall[t, pl.ds(16 * g, 16)]
                ilist[b][pl.ds(16 * g, 16)] = iv

            @pl.when(jnp.asarray(t >= 2))
            def _():
                pltpu.make_async_copy(
                    gbuf[b], out_slice(t - 2), o_s[b]).wait()

            pltpu.async_copy(w_hbm.at[ilist[b]], gbuf[b], g_s[b])

            @pl.when(jnp.asarray(t > 0))
            def _():
                ob = 1 - b
                pltpu.make_async_copy(
                    w_hbm.at[ilist[ob]], gbuf[ob], g_s[ob]).wait()
                pltpu.async_copy(gbuf[ob], out_slice(t - 1), o_s[ob])

        @pl.loop(0, UPW, step=2)
        def _(tt):
            stage(tt, 0)
            stage(tt + 1, 1)

        pltpu.make_async_copy(
            w_hbm.at[ilist[1]], gbuf[1], g_s[1]).wait()
        pltpu.make_async_copy(
            gbuf[0], out_slice(UPW - 2), o_s[0]).wait()
        pltpu.async_copy(gbuf[1], out_slice(UPW - 1), o_s[1])
        pltpu.make_async_copy(
            gbuf[1], out_slice(UPW - 1), o_s[1]).wait()

    return k(w128, idx_t)


def kernel(indices, weight):
    w128 = _tc_in(weight.T)
    idx_t = indices.astype(jnp.int32).T
    stage = _sc_gather(w128, idx_t)
    out2 = _tc_out(stage)
    out3 = out2.reshape(FIELDS, EMB, BATCH)
    return out3.transpose(2, 0, 1)


# speedup vs baseline: 4.4571x; 4.4571x over previous
import functools

import jax
import jax.numpy as jnp
from jax import lax
from jax.experimental import pallas as pl
from jax.experimental.pallas import tpu as pltpu
from jax.experimental.pallas import tpu_sc as plsc

EMB = 64
FIELDS = 26
BATCH = 16384
NBLK = BATCH // 128
NC = 2
NS = 16
NW = NC * NS
CPW = NBLK // NW
UPW = CPW * FIELDS
NROWS = 1000000
TCOL = 4096


def _tc_in_body(x_ref, o_ref):
    xt = x_ref[...].T
    o_ref[...] = jnp.concatenate(
        [xt, jnp.zeros((TCOL, 128 - EMB), jnp.float32)], axis=1)


@jax.jit
def _tc_in(wt):
    grid = (NROWS + TCOL - 1) // TCOL
    return pl.pallas_call(
        _tc_in_body,
        grid=(grid,),
        in_specs=[pl.BlockSpec((EMB, TCOL), lambda i: (0, i))],
        out_specs=pl.BlockSpec((TCOL, 128), lambda i: (i, 0)),
        out_shape=jax.ShapeDtypeStruct((NROWS, 128), jnp.float32),
    )(wt)


OCHUNK = 2048


def _tc_out_body(x_ref, o_ref):
    o_ref[...] = lax.slice(x_ref[...].T, (0, 0), (EMB, OCHUNK))


@jax.jit
def _tc_out(stage):
    nco = BATCH // OCHUNK
    return pl.pallas_call(
        _tc_out_body,
        grid=(FIELDS, nco),
        in_specs=[pl.BlockSpec((OCHUNK, 128), lambda f, c: (f * nco + c, 0))],
        out_specs=pl.BlockSpec((EMB, OCHUNK), lambda f, c: (f, c)),
        out_shape=jax.ShapeDtypeStruct((FIELDS * EMB, BATCH), jnp.float32),
    )(stage)


@jax.jit
def _sc_gather(w128, idx_t):
    mesh = plsc.VectorSubcoreMesh(core_axis_name="c", subcore_axis_name="s")

    @functools.partial(
        pl.kernel,
        out_type=jax.ShapeDtypeStruct((FIELDS * BATCH, 128), jnp.float32),
        mesh=mesh,
        scratch_types=(
            pltpu.VMEM((UPW, 128), jnp.int32),
            pltpu.VMEM((128,), jnp.int32),
            pltpu.VMEM((128,), jnp.int32),
            pltpu.VMEM((128, 128), jnp.float32),
            pltpu.VMEM((128, 128), jnp.float32),
            pltpu.SemaphoreType.DMA,
            pltpu.SemaphoreType.DMA,
            pltpu.SemaphoreType.DMA,
            pltpu.SemaphoreType.DMA,
            pltpu.SemaphoreType.DMA,
        ),
        compiler_params=pltpu.CompilerParams(
            use_tc_tiling_on_sc=True, needs_layout_passes=False),
    )
    def k(w_hbm, idx_hbm, out_hbm, idxall, ilist0, ilist1,
          gbuf0, gbuf1, i_sem, g_s0, g_s1, o_s0, o_s1):
        ilist = (ilist0, ilist1)
        gbuf = (gbuf0, gbuf1)
        g_s = (g_s0, g_s1)
        o_s = (o_s0, o_s1)
        wid = lax.axis_index("s") * NC + lax.axis_index("c")

        for ci in range(CPW):
            col = (wid + NW * ci) * 128
            pltpu.async_copy(
                idx_hbm.at[pl.ds(0, FIELDS), pl.ds(col, 128)],
                idxall.at[pl.ds(ci * FIELDS, FIELDS)], i_sem)
        for ci in range(CPW):
            col = (wid + NW * ci) * 128
            pltpu.make_async_copy(
                idx_hbm.at[pl.ds(0, FIELDS), pl.ds(col, 128)],
                idxall.at[pl.ds(ci * FIELDS, FIELDS)], i_sem).wait()

        def unit_fc(t):
            t = jnp.asarray(t, jnp.int32)
            ci = ((t >= FIELDS).astype(jnp.int32)
                  + (t >= 2 * FIELDS).astype(jnp.int32)
                  + (t >= 3 * FIELDS).astype(jnp.int32))
            f = t - FIELDS * ci
            return ci, f

        def out_slice(t):
            ci, f = unit_fc(t)
            col = (wid + NW * ci) * 128
            return out_hbm.at[pl.ds(f * BATCH + col, 128), pl.ds(0, 128)]

        def stage(t, b):
            for g in range(8):
                iv = idxall[t, pl.ds(16 * g, 16)]
                ilist[b][pl.ds(16 * g, 16)] = iv

            @pl.when(jnp.asarray(t >= 2))
            def _():
                pltpu.make_async_copy(
                    gbuf[b], out_slice(t - 2), o_s[b]).wait()

            pltpu.async_copy(w_hbm.at[ilist[b]], gbuf[b], g_s[b])

            @pl.when(jnp.asarray(t > 0))
            def _():
                ob = 1 - b
                pltpu.make_async_copy(
                    w_hbm.at[ilist[ob]], gbuf[ob], g_s[ob]).wait()
                pltpu.async_copy(gbuf[ob], out_slice(t - 1), o_s[ob])

        @pl.loop(0, UPW, step=2)
        def _(tt):
            stage(tt, 0)
            stage(tt + 1, 1)

        pltpu.make_async_copy(
            w_hbm.at[ilist[1]], gbuf[1], g_s[1]).wait()
        pltpu.make_async_copy(
            gbuf[0], out_slice(UPW - 2), o_s[0]).wait()
        pltpu.async_copy(gbuf[1], out_slice(UPW - 1), o_s[1])
        pltpu.make_async_copy(
            gbuf[1], out_slice(UPW - 1), o_s[1]).wait()

    return k(w128, idx_t)


def kernel(indices, weight):
    w128 = _tc_in(weight.T)
    idx_t = indices.astype(jnp.int32).T
    stage = _sc_gather(w128, idx_t)
    out2 = _tc_out(stage)
    out3 = out2.reshape(FIELDS, EMB, BATCH)
    return out3.transpose(2, 0, 1)
